# baseline (speedup 1.0000x reference)
import jax
import jax.numpy as jnp
from jax import lax
from jax.experimental import pallas as pl
from jax.experimental.pallas import tpu as pltpu
from jax.experimental.pallas import tpu_sc as plsc

N = 10000
E = 320000
D = 128
DE = 16
L = 3
G = 64
BN_EPS = 1e-5

NSUB = 16
NCORE = 2
E_PAD = 327680
EH = E_PAD // 2
NTRASH = 8
NA = N + NTRASH
RPS = EH // NSUB
RB = 32
NBLK = RPS // RB
KG = 8
NGRP = NBLK // KG
ZR = 624
ZR_LAST = NA - (NSUB - 1) * ZR


def _bn_relu_body(h_ref, g_ref, b_ref, o_ref):
    h = h_ref[...]
    mean = jnp.mean(h, axis=0, keepdims=True)
    d = h - mean
    var = jnp.mean(d * d, axis=0, keepdims=True)
    o_ref[...] = jax.nn.relu(d * lax.rsqrt(var + BN_EPS) * g_ref[...]
                             + b_ref[...])


def _bn_relu(h, g, b):
    return pl.pallas_call(
        _bn_relu_body,
        out_shape=jax.ShapeDtypeStruct((N, D), jnp.float32),
    )(h, g.reshape(1, D), b.reshape(1, D))


BE = 4096


def _edge_mlp_body(a1_ref, a2_ref, w_ref, o_ref):
    m1 = jnp.dot(a1_ref[...], w_ref[...], preferred_element_type=jnp.float32)
    m2 = jnp.dot(a2_ref[...], w_ref[...], preferred_element_type=jnp.float32)
    for c in range(NCORE):
        o_ref[c] = jnp.concatenate(
            [m1[:, c * 64:(c + 1) * 64], m2[:, c * 64:(c + 1) * 64]], axis=1)


def _edge_mlp_split(ea_pad, We):
    return pl.pallas_call(
        _edge_mlp_body,
        grid=(EH // BE,),
        in_specs=[
            pl.BlockSpec((BE, DE), lambda i: (i, 0)),
            pl.BlockSpec((BE, DE), lambda i: (i + EH // BE, 0)),
            pl.BlockSpec((DE, D), lambda i: (0, 0)),
        ],
        out_specs=pl.BlockSpec((NCORE, BE, D), lambda i: (0, i, 0)),
        out_shape=jax.ShapeDtypeStruct((NCORE, EH, D), jnp.float32),
    )(ea_pad, ea_pad, We)


def _sc_agg_body(hn_hbm, e_hbm, sp_hbm, dp_hbm, out_hbm, acc_sh,
                 sidx_v, didx_v, e_v, g_v, o_v, sem_i, sem_e, sem_g, sem_s):
    core = lax.axis_index("c")
    sub = lax.axis_index("s")
    lane0 = core * 64

    @pl.loop(0, 2 * RB)
    def _(i):
        for j in range(D // 16):
            o_v[0].at[pl.ds(i, 1), pl.ds(j * 16, 16)][...] = (
                jnp.zeros((1, 16), jnp.float32))

    @pl.when(sub < NSUB - 1)
    def _():
        for k in range(ZR // (2 * RB)):
            pltpu.sync_copy(o_v[0],
                            acc_sh.at[pl.ds(sub * ZR + k * 2 * RB, 2 * RB)])
        rem = ZR % (2 * RB)
        pltpu.sync_copy(o_v[0].at[pl.ds(0, rem)],
                        acc_sh.at[pl.ds(sub * ZR + ZR - rem, rem)])

    @pl.when(sub == NSUB - 1)
    def _():
        base = (NSUB - 1) * ZR
        for k in range(ZR_LAST // (2 * RB)):
            pltpu.sync_copy(o_v[0],
                            acc_sh.at[pl.ds(base + k * 2 * RB, 2 * RB)])
        rem = ZR_LAST % (2 * RB)
        pltpu.sync_copy(o_v[0].at[pl.ds(0, rem)],
                        acc_sh.at[pl.ds(base + ZR_LAST - rem, rem)])

    plsc.subcore_barrier()

    ggrp0 = sub * NGRP
    NQ = NGRP // 2

    def start_idx_group(gq, s):
        pltpu.make_async_copy(sp_hbm.at[ggrp0 + gq], sidx_v[s],
                              sem_i[s]).start()
        pltpu.make_async_copy(dp_hbm.at[ggrp0 + gq], didx_v[s],
                              sem_i[s]).start()

    def wait_idx_group(s):
        pltpu.make_async_copy(sp_hbm.at[0], sidx_v[s], sem_i[s]).wait()
        pltpu.make_async_copy(dp_hbm.at[0], didx_v[s], sem_i[s]).wait()

    def start_e(b, s):
        pltpu.make_async_copy(e_hbm.at[core, pl.ds(sub * RPS + b * RB, RB)],
                              e_v[s], sem_e[s]).start()

    def start_gather(iq, j, s):
        pltpu.make_async_copy(hn_hbm.at[sidx_v[iq].at[j]], g_v[s],
                              sem_g[s]).start()

    def wait_scatter(s):
        pltpu.make_async_copy(o_v[s], acc_sh.at[pl.ds(0, 2 * RB)],
                              sem_s[s]).wait()

    def compute(iq, j, s, guard_first):
        pltpu.make_async_copy(e_hbm.at[core, pl.ds(0, RB)], e_v[s],
                              sem_e[s]).wait()
        pltpu.make_async_copy(hn_hbm.at[sidx_v[iq].at[j]], g_v[s],
                              sem_g[s]).wait()
        if guard_first is None:
            wait_scatter(s)
        else:
            @pl.when(guard_first)
            def _():
                wait_scatter(s)

        @pl.loop(0, RB)
        def _(r):
            for g in range(4):
                lf = pl.ds(g * 16, 16)
                lb = pl.ds(64 + g * 16, 16)
                lh = pl.ds(lane0 + g * 16, 16)
                hf = g_v[s].at[pl.ds(r, 1), lh][...]
                ef = e_v[s].at[pl.ds(r, 1), lf][...]
                msg = jnp.maximum(hf + ef, 0.0)
                ex = jnp.exp(msg)
                o_v[s].at[pl.ds(r, 1), lf][...] = ex
                o_v[s].at[pl.ds(r, 1), lb][...] = ex * msg
                hb = g_v[s].at[pl.ds(RB + r, 1), lh][...]
                eb = e_v[s].at[pl.ds(r, 1), lb][...]
                msgb = jnp.maximum(hb + eb, 0.0)
                exb = jnp.exp(msgb)
                o_v[s].at[pl.ds(RB + r, 1), lf][...] = exb
                o_v[s].at[pl.ds(RB + r, 1), lb][...] = exb * msgb

        pltpu.async_copy(o_v[s], acc_sh.at[didx_v[iq].at[j]], sem_s[s],
                         add=True)

    start_idx_group(0, 0)
    wait_idx_group(0)
    start_e(0, 0)
    start_gather(0, 0, 0)

    @pl.loop(0, NQ)
    def _(q):
        for half in range(2):
            oh = 1 - half
            for j in range(KG):
                s = j % 2
                nx = (j + 1) % 2
                b = (2 * q + half) * KG + j

                if j == 2:
                    if half == 0:
                        start_idx_group(2 * q + 1, 1)
                    else:
                        @pl.when(q < NQ - 1)
                        def _():
                            start_idx_group(2 * q + 2, 0)

                if j < KG - 1:
                    start_e(b + 1, nx)
                    start_gather(half, j + 1, nx)
                elif half == 0:
                    wait_idx_group(1)
                    start_e(b + 1, nx)
                    start_gather(1, 0, nx)
                else:
                    @pl.when(q < NQ - 1)
                    def _():
                        wait_idx_group(0)
                        start_e(b + 1, nx)
                        start_gather(0, 0, nx)

                guard = (q > 0) if (half == 0 and j < 2) else None
                compute(half, j, s, guard)

    wait_scatter(0)
    wait_scatter(1)
    plsc.subcore_barrier()

    @pl.when(sub < NSUB - 1)
    def _():
        pltpu.sync_copy(acc_sh.at[pl.ds(sub * ZR, ZR)],
                        out_hbm.at[core, pl.ds(sub * ZR, ZR)])

    @pl.when(sub == NSUB - 1)
    def _():
        pltpu.sync_copy(acc_sh.at[pl.ds((NSUB - 1) * ZR, ZR_LAST)],
                        out_hbm.at[core, pl.ds((NSUB - 1) * ZR, ZR_LAST)])


def _sc_agg(hn, e_split, spack, dpack):
    mesh = plsc.VectorSubcoreMesh(core_axis_name="c", subcore_axis_name="s")
    run = pl.kernel(
        _sc_agg_body,
        out_type=jax.ShapeDtypeStruct((NCORE, NA, D), jnp.float32),
        mesh=mesh,
        scratch_types=[
            pltpu.VMEM_SHARED((NA, D), jnp.float32),
            [pltpu.VMEM((KG, 2 * RB), jnp.int32) for _ in range(2)],
            [pltpu.VMEM((KG, 2 * RB), jnp.int32) for _ in range(2)],
            [pltpu.VMEM((RB, D), jnp.float32) for _ in range(2)],
            [pltpu.VMEM((2 * RB, D), jnp.float32) for _ in range(2)],
            [pltpu.VMEM((2 * RB, D), jnp.float32) for _ in range(2)],
            [pltpu.SemaphoreType.DMA for _ in range(2)],
            [pltpu.SemaphoreType.DMA for _ in range(2)],
            [pltpu.SemaphoreType.DMA for _ in range(2)],
            [pltpu.SemaphoreType.DMA for _ in range(2)],
        ],
    )
    return run(hn, e_split, spack, dpack)


BR = 2000


def _combine_body(sc_ref, hn_ref, h_ref, w_ref, o_ref):
    den = jnp.concatenate([sc_ref[0, :, :64], sc_ref[1, :, :64]], axis=1)
    num = jnp.concatenate([sc_ref[0, :, 64:], sc_ref[1, :, 64:]], axis=1)
    y = num / jnp.maximum(den, 1e-16) + hn_ref[...]
    o_ref[...] = h_ref[...] + jnp.dot(y, w_ref[...],
                                      preferred_element_type=jnp.float32)


def _combine(sc_out, hn, h, Wm):
    return pl.pallas_call(
        _combine_body,
        grid=(N // BR,),
        in_specs=[
            pl.BlockSpec((NCORE, BR, D), lambda i: (0, i, 0)),
            pl.BlockSpec((BR, D), lambda i: (i, 0)),
            pl.BlockSpec((BR, D), lambda i: (i, 0)),
            pl.BlockSpec((D, D), lambda i: (0, 0)),
        ],
        out_specs=pl.BlockSpec((BR, D), lambda i: (i, 0)),
        out_shape=jax.ShapeDtypeStruct((N, D), jnp.float32),
    )(sc_out, hn, h, Wm)


def _pool_body(h_ref, batch_ref, o_ref):
    gids = lax.broadcasted_iota(jnp.int32, (N, G), 1)
    onehot = (batch_ref[...] == gids).astype(jnp.float32)
    o_ref[...] = lax.dot_general(onehot, h_ref[...], (((0,), (0,)), ((), ())),
                                 preferred_element_type=jnp.float32)


def _pool(h, batch):
    return pl.pallas_call(
        _pool_body,
        out_shape=jax.ShapeDtypeStruct((G, D), jnp.float32),
    )(h, batch.reshape(N, 1))


def _pack_blocks(v):
    front = v[:EH].reshape(NSUB, NBLK, RB)
    back = v[EH:].reshape(NSUB, NBLK, RB)
    return jnp.concatenate([front, back], axis=2).reshape(
        NSUB * NGRP, KG, 2 * RB)


def kernel(x, edge_index, edge_attr, batch, W_mlp, W_edge, gamma, beta):
    src, dst = edge_index[0], edge_index[1]
    npad = E_PAD - E
    srcp = jnp.concatenate([src, jnp.zeros((npad,), jnp.int32)])
    dstp = jnp.concatenate(
        [dst, N + (jnp.arange(npad, dtype=jnp.int32) % NTRASH)])
    spack = _pack_blocks(srcp)
    dpack = _pack_blocks(dstp)
    ea_pad = jnp.concatenate(
        [edge_attr, jnp.zeros((npad, DE), jnp.float32)], axis=0)

    e_splits = [_edge_mlp_split(ea_pad, W_edge[l]) for l in range(L)]

    h = x
    for l in range(L):
        hn = _bn_relu(h, gamma[l], beta[l])
        sc_out = _sc_agg(hn, e_splits[l], spack, dpack)
        h = _combine(sc_out[:, :N, :], hn, h, W_mlp[l])
    hf = _bn_relu(h, gamma[0], beta[0])
    return _pool(hf, batch)


# speedup vs baseline: 1.2891x; 1.2891x over previous
import jax
import jax.numpy as jnp
from jax import lax
from jax.experimental import pallas as pl
from jax.experimental.pallas import tpu as pltpu
from jax.experimental.pallas import tpu_sc as plsc

N = 10000
E = 320000
D = 128
DE = 16
L = 3
G = 64
BN_EPS = 1e-5

NSUB = 16
NCORE = 2
E_PAD = 327680
EH = E_PAD // 2
NTRASH = 8
NA = N + NTRASH
RPS = EH // NSUB
RB = 32
NBLK = RPS // RB
KG = 8
NGRP = NBLK // KG
ZR = 624
ZR_LAST = NA - (NSUB - 1) * ZR


def _bn_relu_body(h_ref, g_ref, b_ref, o_ref):
    h = h_ref[...]
    mean = jnp.mean(h, axis=0, keepdims=True)
    d = h - mean
    var = jnp.mean(d * d, axis=0, keepdims=True)
    o_ref[...] = jax.nn.relu(d * lax.rsqrt(var + BN_EPS) * g_ref[...]
                             + b_ref[...])


def _bn_relu(h, g, b):
    return pl.pallas_call(
        _bn_relu_body,
        out_shape=jax.ShapeDtypeStruct((N, D), jnp.float32),
    )(h, g.reshape(1, D), b.reshape(1, D))


BE = 4096


def _edge_mlp_body(a1_ref, a2_ref, w_ref, o_ref):
    m1 = jnp.dot(a1_ref[...], w_ref[...], preferred_element_type=jnp.float32)
    m2 = jnp.dot(a2_ref[...], w_ref[...], preferred_element_type=jnp.float32)
    for c in range(NCORE):
        o_ref[c] = jnp.concatenate(
            [m1[:, c * 64:(c + 1) * 64], m2[:, c * 64:(c + 1) * 64]], axis=1)


def _edge_mlp_split(ea_pad, We):
    return pl.pallas_call(
        _edge_mlp_body,
        grid=(EH // BE,),
        in_specs=[
            pl.BlockSpec((BE, DE), lambda i: (i, 0)),
            pl.BlockSpec((BE, DE), lambda i: (i + EH // BE, 0)),
            pl.BlockSpec((DE, D), lambda i: (0, 0)),
        ],
        out_specs=pl.BlockSpec((NCORE, BE, D), lambda i: (0, i, 0)),
        out_shape=jax.ShapeDtypeStruct((NCORE, EH, D), jnp.float32),
    )(ea_pad, ea_pad, We)


def _sc_agg_body(hn_hbm, e_hbm, sp_hbm, dp_hbm, out_hbm, acc_sh,
                 sidx_v, didx_v, e_v, g_v, o_v, sem_i, sem_e, sem_g, sem_s):
    core = lax.axis_index("c")
    sub = lax.axis_index("s")
    lane0 = core * 64

    @pl.loop(0, 2 * RB)
    def _(i):
        for j in range(D // 16):
            o_v[0].at[pl.ds(i, 1), pl.ds(j * 16, 16)][...] = (
                jnp.zeros((1, 16), jnp.float32))

    @pl.when(sub < NSUB - 1)
    def _():
        for k in range(ZR // (2 * RB)):
            pltpu.sync_copy(o_v[0],
                            acc_sh.at[pl.ds(sub * ZR + k * 2 * RB, 2 * RB)])
        rem = ZR % (2 * RB)
        pltpu.sync_copy(o_v[0].at[pl.ds(0, rem)],
                        acc_sh.at[pl.ds(sub * ZR + ZR - rem, rem)])

    @pl.when(sub == NSUB - 1)
    def _():
        base = (NSUB - 1) * ZR
        for k in range(ZR_LAST // (2 * RB)):
            pltpu.sync_copy(o_v[0],
                            acc_sh.at[pl.ds(base + k * 2 * RB, 2 * RB)])
        rem = ZR_LAST % (2 * RB)
        pltpu.sync_copy(o_v[0].at[pl.ds(0, rem)],
                        acc_sh.at[pl.ds(base + ZR_LAST - rem, rem)])

    plsc.subcore_barrier()

    ggrp0 = sub * NGRP
    NQ = NGRP // 2

    def start_idx_group(gq, s):
        pltpu.make_async_copy(sp_hbm.at[ggrp0 + gq], sidx_v[s],
                              sem_i[s]).start()
        pltpu.make_async_copy(dp_hbm.at[ggrp0 + gq], didx_v[s],
                              sem_i[s]).start()

    def wait_idx_group(s):
        pltpu.make_async_copy(sp_hbm.at[0], sidx_v[s], sem_i[s]).wait()
        pltpu.make_async_copy(dp_hbm.at[0], didx_v[s], sem_i[s]).wait()

    def start_e(b, s):
        pltpu.make_async_copy(e_hbm.at[core, pl.ds(sub * RPS + b * RB, RB)],
                              e_v[s], sem_e[s]).start()

    def start_gather(iq, j, s):
        pltpu.make_async_copy(hn_hbm.at[sidx_v[iq].at[j]], g_v[s],
                              sem_g[s]).start()

    def wait_scatter(s):
        pltpu.make_async_copy(o_v[s], acc_sh.at[pl.ds(0, 2 * RB)],
                              sem_s[s]).wait()

    def compute(iq, j, s, guard_first):
        pltpu.make_async_copy(e_hbm.at[core, pl.ds(0, RB)], e_v[s],
                              sem_e[s]).wait()
        pltpu.make_async_copy(hn_hbm.at[sidx_v[iq].at[j]], g_v[s],
                              sem_g[s]).wait()
        if guard_first is None:
            wait_scatter(s)
        else:
            @pl.when(guard_first)
            def _():
                wait_scatter(s)

        @plsc.parallel_loop(0, RB, unroll=4)
        def _(r):
            for g in range(4):
                lf = pl.ds(g * 16, 16)
                lb = pl.ds(64 + g * 16, 16)
                lh = pl.ds(lane0 + g * 16, 16)
                hf = g_v[s].at[pl.ds(r, 1), lh][...]
                ef = e_v[s].at[pl.ds(r, 1), lf][...]
                msg = jnp.maximum(hf + ef, 0.0)
                ex = jnp.exp(msg)
                o_v[s].at[pl.ds(r, 1), lf][...] = ex
                o_v[s].at[pl.ds(r, 1), lb][...] = ex * msg
                hb = g_v[s].at[pl.ds(RB + r, 1), lh][...]
                eb = e_v[s].at[pl.ds(r, 1), lb][...]
                msgb = jnp.maximum(hb + eb, 0.0)
                exb = jnp.exp(msgb)
                o_v[s].at[pl.ds(RB + r, 1), lf][...] = exb
                o_v[s].at[pl.ds(RB + r, 1), lb][...] = exb * msgb

        pltpu.async_copy(o_v[s], acc_sh.at[didx_v[iq].at[j]], sem_s[s],
                         add=True)

    start_idx_group(0, 0)
    wait_idx_group(0)
    start_e(0, 0)
    start_gather(0, 0, 0)

    @pl.loop(0, NQ)
    def _(q):
        for half in range(2):
            oh = 1 - half
            for j in range(KG):
                s = j % 2
                nx = (j + 1) % 2
                b = (2 * q + half) * KG + j

                if j == 2:
                    if half == 0:
                        start_idx_group(2 * q + 1, 1)
                    else:
                        @pl.when(q < NQ - 1)
                        def _():
                            start_idx_group(2 * q + 2, 0)

                if j < KG - 1:
                    start_e(b + 1, nx)
                    start_gather(half, j + 1, nx)
                elif half == 0:
                    wait_idx_group(1)
                    start_e(b + 1, nx)
                    start_gather(1, 0, nx)
                else:
                    @pl.when(q < NQ - 1)
                    def _():
                        wait_idx_group(0)
                        start_e(b + 1, nx)
                        start_gather(0, 0, nx)

                guard = (q > 0) if (half == 0 and j < 2) else None
                compute(half, j, s, guard)

    wait_scatter(0)
    wait_scatter(1)
    plsc.subcore_barrier()

    @pl.when(sub < NSUB - 1)
    def _():
        pltpu.sync_copy(acc_sh.at[pl.ds(sub * ZR, ZR)],
                        out_hbm.at[core, pl.ds(sub * ZR, ZR)])

    @pl.when(sub == NSUB - 1)
    def _():
        pltpu.sync_copy(acc_sh.at[pl.ds((NSUB - 1) * ZR, ZR_LAST)],
                        out_hbm.at[core, pl.ds((NSUB - 1) * ZR, ZR_LAST)])


def _sc_agg(hn, e_split, spack, dpack):
    mesh = plsc.VectorSubcoreMesh(core_axis_name="c", subcore_axis_name="s")
    run = pl.kernel(
        _sc_agg_body,
        out_type=jax.ShapeDtypeStruct((NCORE, NA, D), jnp.float32),
        mesh=mesh,
        scratch_types=[
            pltpu.VMEM_SHARED((NA, D), jnp.float32),
            [pltpu.VMEM((KG, 2 * RB), jnp.int32) for _ in range(2)],
            [pltpu.VMEM((KG, 2 * RB), jnp.int32) for _ in range(2)],
            [pltpu.VMEM((RB, D), jnp.float32) for _ in range(2)],
            [pltpu.VMEM((2 * RB, D), jnp.float32) for _ in range(2)],
            [pltpu.VMEM((2 * RB, D), jnp.float32) for _ in range(2)],
            [pltpu.SemaphoreType.DMA for _ in range(2)],
            [pltpu.SemaphoreType.DMA for _ in range(2)],
            [pltpu.SemaphoreType.DMA for _ in range(2)],
            [pltpu.SemaphoreType.DMA for _ in range(2)],
        ],
    )
    return run(hn, e_split, spack, dpack)


BR = 2000


def _combine_body(sc_ref, hn_ref, h_ref, w_ref, o_ref):
    den = jnp.concatenate([sc_ref[0, :, :64], sc_ref[1, :, :64]], axis=1)
    num = jnp.concatenate([sc_ref[0, :, 64:], sc_ref[1, :, 64:]], axis=1)
    y = num / jnp.maximum(den, 1e-16) + hn_ref[...]
    o_ref[...] = h_ref[...] + jnp.dot(y, w_ref[...],
                                      preferred_element_type=jnp.float32)


def _combine(sc_out, hn, h, Wm):
    return pl.pallas_call(
        _combine_body,
        grid=(N // BR,),
        in_specs=[
            pl.BlockSpec((NCORE, BR, D), lambda i: (0, i, 0)),
            pl.BlockSpec((BR, D), lambda i: (i, 0)),
            pl.BlockSpec((BR, D), lambda i: (i, 0)),
            pl.BlockSpec((D, D), lambda i: (0, 0)),
        ],
        out_specs=pl.BlockSpec((BR, D), lambda i: (i, 0)),
        out_shape=jax.ShapeDtypeStruct((N, D), jnp.float32),
    )(sc_out, hn, h, Wm)


def _pool_body(h_ref, batch_ref, o_ref):
    gids = lax.broadcasted_iota(jnp.int32, (N, G), 1)
    onehot = (batch_ref[...] == gids).astype(jnp.float32)
    o_ref[...] = lax.dot_general(onehot, h_ref[...], (((0,), (0,)), ((), ())),
                                 preferred_element_type=jnp.float32)


def _pool(h, batch):
    return pl.pallas_call(
        _pool_body,
        out_shape=jax.ShapeDtypeStruct((G, D), jnp.float32),
    )(h, batch.reshape(N, 1))


def _pack_blocks(v):
    front = v[:EH].reshape(NSUB, NBLK, RB)
    back = v[EH:].reshape(NSUB, NBLK, RB)
    return jnp.concatenate([front, back], axis=2).reshape(
        NSUB * NGRP, KG, 2 * RB)


def kernel(x, edge_index, edge_attr, batch, W_mlp, W_edge, gamma, beta):
    src, dst = edge_index[0], edge_index[1]
    npad = E_PAD - E
    srcp = jnp.concatenate([src, jnp.zeros((npad,), jnp.int32)])
    dstp = jnp.concatenate(
        [dst, N + (jnp.arange(npad, dtype=jnp.int32) % NTRASH)])
    spack = _pack_blocks(srcp)
    dpack = _pack_blocks(dstp)
    ea_pad = jnp.concatenate(
        [edge_attr, jnp.zeros((npad, DE), jnp.float32)], axis=0)

    e_splits = [_edge_mlp_split(ea_pad, W_edge[l]) for l in range(L)]

    h = x
    for l in range(L):
        hn = _bn_relu(h, gamma[l], beta[l])
        sc_out = _sc_agg(hn, e_splits[l], spack, dpack)
        h = _combine(sc_out[:, :N, :], hn, h, W_mlp[l])
    hf = _bn_relu(h, gamma[0], beta[0])
    return _pool(hf, batch)


# speedup vs baseline: 1.2916x; 1.0019x over previous
import jax
import jax.numpy as jnp
from jax import lax
from jax.experimental import pallas as pl
from jax.experimental.pallas import tpu as pltpu
from jax.experimental.pallas import tpu_sc as plsc

N = 10000
E = 320000
D = 128
DE = 16
L = 3
G = 64
BN_EPS = 1e-5

NSUB = 16
NCORE = 2
E_PAD = 327680
EH = E_PAD // 2
NTRASH = 8
NA = N + NTRASH
RPS = EH // NSUB
RB = 32
NBLK = RPS // RB
KG = 8
NGRP = NBLK // KG
ZR = 624
ZR_LAST = NA - (NSUB - 1) * ZR


def _bn_relu_body(h_ref, g_ref, b_ref, o_ref):
    h = h_ref[...]
    mean = jnp.mean(h, axis=0, keepdims=True)
    d = h - mean
    var = jnp.mean(d * d, axis=0, keepdims=True)
    o_ref[...] = jax.nn.relu(d * lax.rsqrt(var + BN_EPS) * g_ref[...]
                             + b_ref[...])


def _bn_relu(h, g, b):
    return pl.pallas_call(
        _bn_relu_body,
        out_shape=jax.ShapeDtypeStruct((N, D), jnp.float32),
    )(h, g.reshape(1, D), b.reshape(1, D))


BE = 4096


def _edge_mlp_body(a1_ref, a2_ref, w_ref, o_ref):
    m1 = jnp.dot(a1_ref[...], w_ref[...], preferred_element_type=jnp.float32)
    m2 = jnp.dot(a2_ref[...], w_ref[...], preferred_element_type=jnp.float32)
    for c in range(NCORE):
        o_ref[c] = jnp.concatenate(
            [m1[:, c * 64:(c + 1) * 64], m2[:, c * 64:(c + 1) * 64]], axis=1)


def _edge_mlp_split(ea_pad, We):
    return pl.pallas_call(
        _edge_mlp_body,
        grid=(EH // BE,),
        in_specs=[
            pl.BlockSpec((BE, DE), lambda i: (i, 0)),
            pl.BlockSpec((BE, DE), lambda i: (i + EH // BE, 0)),
            pl.BlockSpec((DE, D), lambda i: (0, 0)),
        ],
        out_specs=pl.BlockSpec((NCORE, BE, D), lambda i: (0, i, 0)),
        out_shape=jax.ShapeDtypeStruct((NCORE, EH, D), jnp.float32),
    )(ea_pad, ea_pad, We)


def _sc_agg_body(hn_hbm, e_hbm, sp_hbm, dp_hbm, out_hbm, acc_sh,
                 sidx_v, didx_v, e_v, g_v, o_v, sem_i, sem_e, sem_g, sem_s):
    core = lax.axis_index("c")
    sub = lax.axis_index("s")
    lane0 = core * 64

    @pl.loop(0, 2 * RB)
    def _(i):
        for j in range(D // 16):
            o_v[0].at[pl.ds(i, 1), pl.ds(j * 16, 16)][...] = (
                jnp.zeros((1, 16), jnp.float32))

    @pl.when(sub < NSUB - 1)
    def _():
        for k in range(ZR // (2 * RB)):
            pltpu.sync_copy(o_v[0],
                            acc_sh.at[pl.ds(sub * ZR + k * 2 * RB, 2 * RB)])
        rem = ZR % (2 * RB)
        pltpu.sync_copy(o_v[0].at[pl.ds(0, rem)],
                        acc_sh.at[pl.ds(sub * ZR + ZR - rem, rem)])

    @pl.when(sub == NSUB - 1)
    def _():
        base = (NSUB - 1) * ZR
        for k in range(ZR_LAST // (2 * RB)):
            pltpu.sync_copy(o_v[0],
                            acc_sh.at[pl.ds(base + k * 2 * RB, 2 * RB)])
        rem = ZR_LAST % (2 * RB)
        pltpu.sync_copy(o_v[0].at[pl.ds(0, rem)],
                        acc_sh.at[pl.ds(base + ZR_LAST - rem, rem)])

    plsc.subcore_barrier()

    ggrp0 = sub * NGRP
    NQ = NGRP // 2

    def start_idx_group(gq, s):
        pltpu.make_async_copy(sp_hbm.at[ggrp0 + gq], sidx_v[s],
                              sem_i[s]).start()
        pltpu.make_async_copy(dp_hbm.at[ggrp0 + gq], didx_v[s],
                              sem_i[s]).start()

    def wait_idx_group(s):
        pltpu.make_async_copy(sp_hbm.at[0], sidx_v[s], sem_i[s]).wait()
        pltpu.make_async_copy(dp_hbm.at[0], didx_v[s], sem_i[s]).wait()

    def start_e(b, s):
        pltpu.make_async_copy(e_hbm.at[core, pl.ds(sub * RPS + b * RB, RB)],
                              e_v[s], sem_e[s]).start()

    def start_gather(iq, j, s):
        pltpu.make_async_copy(hn_hbm.at[sidx_v[iq].at[j]], g_v[s],
                              sem_g[s]).start()

    def wait_scatter(s):
        pltpu.make_async_copy(o_v[s], acc_sh.at[pl.ds(0, 2 * RB)],
                              sem_s[s]).wait()

    def compute(iq, j, s, guard_first):
        pltpu.make_async_copy(e_hbm.at[core, pl.ds(0, RB)], e_v[s],
                              sem_e[s]).wait()
        pltpu.make_async_copy(hn_hbm.at[sidx_v[iq].at[j]], g_v[s],
                              sem_g[s]).wait()
        if guard_first is None:
            wait_scatter(s)
        else:
            @pl.when(guard_first)
            def _():
                wait_scatter(s)

        @plsc.parallel_loop(0, RB, unroll=4)
        def _(r):
            for g in range(4):
                lf = pl.ds(g * 16, 16)
                lb = pl.ds(64 + g * 16, 16)
                lh = pl.ds(lane0 + g * 16, 16)
                hf = g_v[s].at[pl.ds(r, 1), lh][...]
                ef = e_v[s].at[pl.ds(r, 1), lf][...]
                msg = jnp.maximum(hf + ef, 0.0)
                ex = jnp.exp(msg)
                o_v[s].at[pl.ds(r, 1), lf][...] = ex
                o_v[s].at[pl.ds(r, 1), lb][...] = ex * msg
                hb = g_v[s].at[pl.ds(RB + r, 1), lh][...]
                eb = e_v[s].at[pl.ds(r, 1), lb][...]
                msgb = jnp.maximum(hb + eb, 0.0)
                exb = jnp.exp(msgb)
                o_v[s].at[pl.ds(RB + r, 1), lf][...] = exb
                o_v[s].at[pl.ds(RB + r, 1), lb][...] = exb * msgb

        pltpu.async_copy(o_v[s], acc_sh.at[didx_v[iq].at[j]], sem_s[s],
                         add=True)

    start_idx_group(0, 0)
    wait_idx_group(0)
    start_e(0, 0)
    start_gather(0, 0, 0)

    @pl.loop(0, NQ)
    def _(q):
        for half in range(2):
            oh = 1 - half
            for j in range(KG):
                s = j % 2
                nx = (j + 1) % 2
                b = (2 * q + half) * KG + j

                if j == 2:
                    if half == 0:
                        start_idx_group(2 * q + 1, 1)
                    else:
                        @pl.when(q < NQ - 1)
                        def _():
                            start_idx_group(2 * q + 2, 0)

                if j < KG - 1:
                    start_e(b + 1, nx)
                    start_gather(half, j + 1, nx)
                elif half == 0:
                    wait_idx_group(1)
                    start_e(b + 1, nx)
                    start_gather(1, 0, nx)
                else:
                    @pl.when(q < NQ - 1)
                    def _():
                        wait_idx_group(0)
                        start_e(b + 1, nx)
                        start_gather(0, 0, nx)

                guard = (q > 0) if (half == 0 and j < 2) else None
                compute(half, j, s, guard)

    wait_scatter(0)
    wait_scatter(1)
    plsc.subcore_barrier()

    @pl.when(sub < NSUB - 1)
    def _():
        pltpu.sync_copy(acc_sh.at[pl.ds(sub * ZR, ZR)],
                        out_hbm.at[core, pl.ds(sub * ZR, ZR)])

    @pl.when(sub == NSUB - 1)
    def _():
        pltpu.sync_copy(acc_sh.at[pl.ds((NSUB - 1) * ZR, ZR_LAST)],
                        out_hbm.at[core, pl.ds((NSUB - 1) * ZR, ZR_LAST)])


def _sc_agg(hn, e_split, spack, dpack):
    mesh = plsc.VectorSubcoreMesh(core_axis_name="c", subcore_axis_name="s")
    run = pl.kernel(
        _sc_agg_body,
        out_type=jax.ShapeDtypeStruct((NCORE, NA, D), jnp.float32),
        mesh=mesh,
        scratch_types=[
            pltpu.VMEM_SHARED((NA, D), jnp.float32),
            [pltpu.VMEM((KG, 2 * RB), jnp.int32) for _ in range(2)],
            [pltpu.VMEM((KG, 2 * RB), jnp.int32) for _ in range(2)],
            [pltpu.VMEM((RB, D), jnp.float32) for _ in range(2)],
            [pltpu.VMEM((2 * RB, D), jnp.float32) for _ in range(2)],
            [pltpu.VMEM((2 * RB, D), jnp.float32) for _ in range(2)],
            [pltpu.SemaphoreType.DMA for _ in range(2)],
            [pltpu.SemaphoreType.DMA for _ in range(2)],
            [pltpu.SemaphoreType.DMA for _ in range(2)],
            [pltpu.SemaphoreType.DMA for _ in range(2)],
        ],
    )
    return run(hn, e_split, spack, dpack)


def _assemble(sc_ref, hn_ref, h_ref, w_ref):
    sc = sc_ref[...]
    den = jnp.concatenate([sc[0, :N, :64], sc[1, :N, :64]], axis=1)
    num = jnp.concatenate([sc[0, :N, 64:], sc[1, :N, 64:]], axis=1)
    y = num / jnp.maximum(den, 1e-16) + hn_ref[...]
    return h_ref[...] + jnp.dot(y, w_ref[...],
                                preferred_element_type=jnp.float32)


def _bn_stats_relu(hnew, g_ref, b_ref):
    mean = jnp.mean(hnew, axis=0, keepdims=True)
    d = hnew - mean
    var = jnp.mean(d * d, axis=0, keepdims=True)
    return jax.nn.relu(d * lax.rsqrt(var + BN_EPS) * g_ref[...] + b_ref[...])


def _combine_bn_body(sc_ref, hn_ref, h_ref, w_ref, g_ref, b_ref,
                     oh_ref, ohn_ref):
    hnew = _assemble(sc_ref, hn_ref, h_ref, w_ref)
    oh_ref[...] = hnew
    ohn_ref[...] = _bn_stats_relu(hnew, g_ref, b_ref)


def _combine_bn(sc_out, hn, h, Wm, g_next, b_next):
    return pl.pallas_call(
        _combine_bn_body,
        out_shape=[jax.ShapeDtypeStruct((N, D), jnp.float32),
                   jax.ShapeDtypeStruct((N, D), jnp.float32)],
    )(sc_out, hn, h, Wm, g_next.reshape(1, D), b_next.reshape(1, D))


def _combine_bn_pool_body(sc_ref, hn_ref, h_ref, w_ref, g_ref, b_ref,
                          batch_ref, o_ref):
    hnew = _assemble(sc_ref, hn_ref, h_ref, w_ref)
    hf = _bn_stats_relu(hnew, g_ref, b_ref)
    gids = lax.broadcasted_iota(jnp.int32, (N, G), 1)
    onehot = (batch_ref[...] == gids).astype(jnp.float32)
    o_ref[...] = lax.dot_general(onehot, hf, (((0,), (0,)), ((), ())),
                                 preferred_element_type=jnp.float32)


def _combine_bn_pool(sc_out, hn, h, Wm, g0, b0, batch):
    return pl.pallas_call(
        _combine_bn_pool_body,
        out_shape=jax.ShapeDtypeStruct((G, D), jnp.float32),
    )(sc_out, hn, h, Wm, g0.reshape(1, D), b0.reshape(1, D),
      batch.reshape(N, 1))


def _pack_blocks(v):
    front = v[:EH].reshape(NSUB, NBLK, RB)
    back = v[EH:].reshape(NSUB, NBLK, RB)
    return jnp.concatenate([front, back], axis=2).reshape(
        NSUB * NGRP, KG, 2 * RB)


def kernel(x, edge_index, edge_attr, batch, W_mlp, W_edge, gamma, beta):
    src, dst = edge_index[0], edge_index[1]
    npad = E_PAD - E
    srcp = jnp.concatenate([src, jnp.zeros((npad,), jnp.int32)])
    dstp = jnp.concatenate(
        [dst, N + (jnp.arange(npad, dtype=jnp.int32) % NTRASH)])
    spack = _pack_blocks(srcp)
    dpack = _pack_blocks(dstp)
    ea_pad = jnp.concatenate(
        [edge_attr, jnp.zeros((npad, DE), jnp.float32)], axis=0)

    e_splits = [_edge_mlp_split(ea_pad, W_edge[l]) for l in range(L)]

    h = x
    hn = _bn_relu(h, gamma[0], beta[0])
    for l in range(L):
        sc_out = _sc_agg(hn, e_splits[l], spack, dpack)
        if l < L - 1:
            h, hn = _combine_bn(sc_out, hn, h, W_mlp[l],
                                gamma[l + 1], beta[l + 1])
        else:
            return _combine_bn_pool(sc_out, hn, h, W_mlp[l],
                                    gamma[0], beta[0], batch)
